# baseline (speedup 1.0000x reference)
import functools
import math

import jax
import jax.numpy as jnp
from jax import lax
from jax.experimental import pallas as pl
from jax.experimental.pallas import tpu as pltpu
from jax.experimental.pallas import tpu_sc as plsc

D_MODEL = 64
SCALE = math.sqrt(D_MODEL)

_info = plsc.get_sparse_core_info()
_NC = _info.num_cores
_NS = _info.num_subcores
_L = _info.num_lanes
_NW = _NC * _NS

_NG = 4
_NO = 2
_C = 128


@jax.jit
def _embed(xT, lut_p):
    T, NB = xT.shape
    n_chunks = T
    mesh = plsc.VectorSubcoreMesh(core_axis_name="c", subcore_axis_name="s")

    @functools.partial(
        pl.kernel,
        mesh=mesh,
        out_type=jax.ShapeDtypeStruct((T, D_MODEL, NB), jnp.float32),
        scratch_types=(
            [pltpu.VMEM((T, _C), jnp.int32),
             pltpu.VMEM((_NG, _C), jnp.int32),
             pltpu.VMEM((_NG * _C, _C), jnp.float32),
             pltpu.VMEM((_NO * D_MODEL, _C), jnp.float32)]
            + [pltpu.SemaphoreType.DMA] * (_NG + _NO)
        ),
        compiler_params=pltpu.CompilerParams(needs_layout_passes=False),
    )
    def k(xT_hbm, table_hbm, out_hbm, idx_all, pidx, pairs, outb, *sems):
        gsems = sems[:_NG]
        wsems = sems[_NG:]
        wid = lax.axis_index("s") * _NC + lax.axis_index("c")
        b_base = wid * _C

        pltpu.sync_copy(xT_hbm.at[:, pl.ds(b_base, _C)], idx_all)

        def compute_pidx(t, slot):
            for g in range(_C // _L):
                sl = pl.ds(g * _L, _L)
                pidx[slot, sl] = lax.shift_right_logical(idx_all[t, sl], 1)

        def start_gather(gb):
            pltpu.async_copy(
                table_hbm.at[pidx.at[gb]],
                pairs.at[pl.ds(gb * _C, _C)], gsems[gb])

        def wait_gather(gb):
            pltpu.make_async_copy(
                table_hbm.at[pl.ds(0, _C)],
                pairs.at[pl.ds(gb * _C, _C)], gsems[gb]).wait()

        def start_wb(t, ob):
            pltpu.async_copy(
                outb.at[pl.ds(ob * D_MODEL, D_MODEL)],
                out_hbm.at[t, :, pl.ds(b_base, _C)], wsems[ob])

        def wait_wb(ob):
            pltpu.make_async_copy(
                outb.at[pl.ds(ob * D_MODEL, D_MODEL)],
                out_hbm.at[0, :, pl.ds(b_base, _C)], wsems[ob]).wait()

        iota = lax.iota(jnp.int32, _L)

        def produce(t, gb, ob):
            for g in range(_C // _L):
                sl = pl.ds(g * _L, _L)
                v = idx_all[t, sl]
                colbase = lax.shift_left(lax.bitwise_and(v, 1), 6)
                row = iota + (g * _L + gb * _C)

                def dbody(d, carry):
                    vals = plsc.load_gather(pairs, [row, colbase + d])
                    outb[ob * D_MODEL + d, sl] = vals * SCALE
                    return carry

                lax.fori_loop(0, D_MODEL, dbody, 0, unroll=4)

        for b in range(_NG):
            compute_pidx(b, b)
            start_gather(b)

        for b in range(_NG):
            wait_gather(b)
            if b >= _NO:
                wait_wb(b % _NO)
            produce(b, b, b % _NO)
            compute_pidx(b + _NG, b)
            start_gather(b)
            start_wb(b, b % _NO)

        def outer(gq, carry):
            for b in range(_NG):
                t = gq * _NG + b
                wait_gather(b)
                wait_wb(b % _NO)
                produce(t, b, b % _NO)
                compute_pidx(t + _NG, b)
                start_gather(b)
                start_wb(t, b % _NO)
            return carry

        lax.fori_loop(1, n_chunks // _NG - 1, outer, 0)

        for b in range(_NG):
            t = n_chunks - _NG + b
            wait_gather(b)
            wait_wb(b % _NO)
            produce(t, b, b % _NO)
            start_wb(t, b % _NO)

        for ob in range(_NO):
            wait_wb(ob)

    return k(xT, lut_p)


def kernel(x, lut):
    lut_p = lut.reshape(lut.shape[0] // 2, 2 * D_MODEL)
    xT = x.T
    out_p = _embed(xT, lut_p)
    return jnp.transpose(out_p, (2, 0, 1))
